# Initial kernel scaffold; baseline (speedup 1.0000x reference)
import jax
import jax.numpy as jnp
from jax.experimental import pallas as pl


def kernel(x, x_scale, fine2coarse_index, distances, W1, b1, W2, b2):
    raise NotImplementedError("write your pallas kernel here")


# speedup vs baseline: 1.6071x; 1.6071x over previous
import functools

import jax
import jax.numpy as jnp
from jax import lax
from jax.experimental import pallas as pl
from jax.experimental.pallas import tpu as pltpu
from jax.experimental.pallas import tpu_sc as plsc

NC = 2
NS = 16
NW = NC * NS
BS = 128


def _sc_gather(table, idx3, nblk_per_w, h):
    mesh = plsc.VectorSubcoreMesh(core_axis_name="c", subcore_axis_name="s")
    total = NW * nblk_per_w * BS

    @functools.partial(
        pl.kernel,
        out_type=jax.ShapeDtypeStruct((total, h), jnp.float32),
        mesh=mesh,
        scratch_types=[
            pltpu.VMEM((nblk_per_w, BS), jnp.int32),
            pltpu.VMEM((BS, h), jnp.float32),
            pltpu.SemaphoreType.DMA,
        ],
    )
    def k(table_hbm, idx_hbm, out_hbm, idx_v, buf_v, sem):
        wid = lax.axis_index("s") * NC + lax.axis_index("c")
        pltpu.sync_copy(idx_hbm.at[wid], idx_v)

        def blk_body(j, carry):
            pltpu.async_copy(table_hbm.at[idx_v.at[j]], buf_v, sem).wait()
            pltpu.sync_copy(buf_v, out_hbm.at[pl.ds((j * NW + wid) * BS, BS)])
            return carry

        lax.fori_loop(0, nblk_per_w, blk_body, 0)

    return k(table, idx3)


def _mlp_body(x_ref, g_ref, d_ref, w1a_ref, w1b_ref, b1_ref, w2_ref, b2_ref,
              o_ref):
    inv = 1.0 / (d_ref[...] + 1e-8)
    h = jnp.dot(x_ref[0], w1a_ref[...], preferred_element_type=jnp.float32)
    h = h + jnp.dot(g_ref[...] * inv, w1b_ref[...],
                    preferred_element_type=jnp.float32)
    h = jnp.maximum(h + b1_ref[...], 0.0)
    o_ref[0] = (jnp.dot(h, w2_ref[...], preferred_element_type=jnp.float32)
                + b2_ref[...])


def kernel(x, x_scale, fine2coarse_index, distances, W1, b1, W2, b2):
    B, N, H = x.shape
    R = B * N
    nblk = -(-R // BS)
    nblk = -(-nblk // NW) * NW
    Rpad = nblk * BS
    nblk_per_w = nblk // NW

    idx = fine2coarse_index.astype(jnp.int32)
    offs = (jnp.arange(B, dtype=jnp.int32) * N)[:, None]
    idx_all = (idx[None, :] + offs).reshape(R)
    idx_pad = jnp.concatenate([idx_all, jnp.zeros(Rpad - R, jnp.int32)])
    idx3 = idx_pad.reshape(nblk_per_w, NW, BS).transpose(1, 0, 2)
    table = x_scale.reshape(R, H)

    gathered = _sc_gather(table, idx3, nblk_per_w, H)

    TN = 1000
    nb = N // TN
    out = pl.pallas_call(
        _mlp_body,
        grid=(B, nb),
        in_specs=[
            pl.BlockSpec((1, TN, H), lambda b, n: (b, n, 0)),
            pl.BlockSpec((TN, H), lambda b, n: (b * nb + n, 0)),
            pl.BlockSpec((TN, 1), lambda b, n: (n, 0)),
            pl.BlockSpec((H, H), lambda b, n: (0, 0)),
            pl.BlockSpec((H, H), lambda b, n: (0, 0)),
            pl.BlockSpec((1, H), lambda b, n: (0, 0)),
            pl.BlockSpec((H, H), lambda b, n: (0, 0)),
            pl.BlockSpec((1, H), lambda b, n: (0, 0)),
        ],
        out_specs=pl.BlockSpec((1, TN, H), lambda b, n: (b, n, 0)),
        out_shape=jax.ShapeDtypeStruct((B, N, H), jnp.float32),
        compiler_params=pltpu.CompilerParams(
            dimension_semantics=("parallel", "parallel")),
    )(x, gathered, distances.reshape(N, 1), W1[:H], W1[H:],
      b1.reshape(1, H), W2, b2.reshape(1, H))
    return out


# speedup vs baseline: 1.6521x; 1.0280x over previous
import functools

import jax
import jax.numpy as jnp
from jax import lax
from jax.experimental import pallas as pl
from jax.experimental.pallas import tpu as pltpu
from jax.experimental.pallas import tpu_sc as plsc

NC = 2
NS = 16
NW = NC * NS
BS = 128


def _sc_gather(table, idx3, nblk_per_w, h):
    mesh = plsc.VectorSubcoreMesh(core_axis_name="c", subcore_axis_name="s")
    total = NW * nblk_per_w * BS

    @functools.partial(
        pl.kernel,
        out_type=jax.ShapeDtypeStruct((total, h), jnp.float32),
        mesh=mesh,
        scratch_types=[
            pltpu.VMEM((nblk_per_w, BS), jnp.int32),
            pltpu.VMEM((BS, h), jnp.float32),
            pltpu.VMEM((BS, h), jnp.float32),
            pltpu.SemaphoreType.DMA,
            pltpu.SemaphoreType.DMA,
        ],
    )
    def k(table_hbm, idx_hbm, out_hbm, idx_v, buf0, buf1, sem0, sem1):
        wid = lax.axis_index("s") * NC + lax.axis_index("c")
        pltpu.sync_copy(idx_hbm.at[wid], idx_v)

        def start(j, buf, sem):
            pltpu.async_copy(table_hbm.at[idx_v.at[j]], buf, sem)

        def wait_store(j, buf, sem):
            pltpu.make_async_copy(table_hbm.at[idx_v.at[0]], buf, sem).wait()
            pltpu.sync_copy(buf, out_hbm.at[pl.ds((j * NW + wid) * BS, BS)])

        start(0, buf0, sem0)

        @pl.loop(0, nblk_per_w - 1, step=2)
        def _(j0):
            start(j0 + 1, buf1, sem1)
            wait_store(j0, buf0, sem0)
            start(j0 + 2, buf0, sem0)
            wait_store(j0 + 1, buf1, sem1)

        wait_store(nblk_per_w - 1, buf0, sem0)

    return k(table, idx3)


def _mlp_body(x_ref, g_ref, d_ref, w1a_ref, w1b_ref, b1_ref, w2_ref, b2_ref,
              o_ref):
    inv = 1.0 / (d_ref[...] + 1e-8)
    h = jnp.dot(x_ref[0], w1a_ref[...], preferred_element_type=jnp.float32)
    h = h + jnp.dot(g_ref[...] * inv, w1b_ref[...],
                    preferred_element_type=jnp.float32)
    h = jnp.maximum(h + b1_ref[...], 0.0)
    o_ref[0] = (jnp.dot(h, w2_ref[...], preferred_element_type=jnp.float32)
                + b2_ref[...])


def kernel(x, x_scale, fine2coarse_index, distances, W1, b1, W2, b2):
    B, N, H = x.shape
    R = B * N
    nblk = -(-R // BS)
    nblk = -(-nblk // NW) * NW
    Rpad = nblk * BS
    nblk_per_w = nblk // NW

    idx = fine2coarse_index.astype(jnp.int32)
    offs = (jnp.arange(B, dtype=jnp.int32) * N)[:, None]
    idx_all = (idx[None, :] + offs).reshape(R)
    idx_pad = jnp.concatenate([idx_all, jnp.zeros(Rpad - R, jnp.int32)])
    idx3 = idx_pad.reshape(nblk_per_w, NW, BS).transpose(1, 0, 2)
    table = x_scale.reshape(R, H)

    gathered = _sc_gather(table, idx3, nblk_per_w, H)

    TN = 1000
    nb = N // TN
    out = pl.pallas_call(
        _mlp_body,
        grid=(B, nb),
        in_specs=[
            pl.BlockSpec((1, TN, H), lambda b, n: (b, n, 0)),
            pl.BlockSpec((TN, H), lambda b, n: (b * nb + n, 0)),
            pl.BlockSpec((TN, 1), lambda b, n: (n, 0)),
            pl.BlockSpec((H, H), lambda b, n: (0, 0)),
            pl.BlockSpec((H, H), lambda b, n: (0, 0)),
            pl.BlockSpec((1, H), lambda b, n: (0, 0)),
            pl.BlockSpec((H, H), lambda b, n: (0, 0)),
            pl.BlockSpec((1, H), lambda b, n: (0, 0)),
        ],
        out_specs=pl.BlockSpec((1, TN, H), lambda b, n: (b, n, 0)),
        out_shape=jax.ShapeDtypeStruct((B, N, H), jnp.float32),
        compiler_params=pltpu.CompilerParams(
            dimension_semantics=("parallel", "parallel")),
    )(x, gathered, distances.reshape(N, 1), W1[:H], W1[H:],
      b1.reshape(1, H), W2, b2.reshape(1, H))
    return out
